# baseline (speedup 1.0000x reference)
import functools

import jax
import jax.numpy as jnp
from jax import lax
from jax.experimental import pallas as pl
from jax.experimental.pallas import tpu as pltpu
from jax.experimental.pallas import tpu_sc as plsc

BATCH = 16384
N_COLS = 1000
K = 256
THRESHOLD = 0.5

NUM_CORES = 2
NUM_SUBCORES = 16
NUM_WORKERS = NUM_CORES * NUM_SUBCORES
ROWS_PER_WORKER = BATCH // NUM_WORKERS
CHUNK = 64
NUM_CHUNKS = ROWS_PER_WORKER // CHUNK
LANES = 16
K_VECS = K // LANES


def _sc_kernel(sim_hbm, idx_hbm, out_hbm, idx_v, rows_v, out_v):
    wid = lax.axis_index("s") * NUM_CORES + lax.axis_index("c")
    base = wid * ROWS_PER_WORKER

    pltpu.sync_copy(idx_hbm, idx_v)
    idx_vecs = [idx_v[pl.ds(j * LANES, LANES)] for j in range(K_VECS)]

    def chunk_body(c, carry):
        row0 = base + c * CHUNK
        pltpu.sync_copy(sim_hbm.at[pl.ds(row0 * N_COLS, CHUNK * N_COLS)], rows_v)

        def row_body(r, carry2):
            rbase = jnp.full((LANES,), r * N_COLS, jnp.int32)
            for j in range(K_VECS):
                g = plsc.load_gather(rows_v, [rbase + idx_vecs[j]])
                out_v[pl.ds(r * K + j * LANES, LANES)] = jnp.where(
                    g >= THRESHOLD, jnp.float32(1.0), jnp.float32(0.0)
                )
            return carry2

        lax.fori_loop(0, CHUNK, row_body, 0)
        pltpu.sync_copy(out_v, out_hbm.at[pl.ds(row0 * K, CHUNK * K)])
        return carry

    lax.fori_loop(0, NUM_CHUNKS, chunk_body, 0)


_call = functools.partial(
    pl.kernel,
    out_type=jax.ShapeDtypeStruct((BATCH * K,), jnp.float32),
    mesh=plsc.VectorSubcoreMesh(core_axis_name="c", subcore_axis_name="s"),
    compiler_params=pltpu.CompilerParams(
        use_tc_tiling_on_sc=False, needs_layout_passes=False
    ),
    scratch_types=[
        pltpu.VMEM((K,), jnp.int32),
        pltpu.VMEM((CHUNK * N_COLS,), jnp.float32),
        pltpu.VMEM((CHUNK * K,), jnp.float32),
    ],
)(_sc_kernel)


def kernel(similarities, label_indices):
    out_flat = _call(similarities.reshape(-1), label_indices)
    return out_flat.reshape(BATCH, K)


# speedup vs baseline: 1.1090x; 1.1090x over previous
import functools

import jax
import jax.numpy as jnp
from jax import lax
from jax.experimental import pallas as pl
from jax.experimental.pallas import tpu as pltpu
from jax.experimental.pallas import tpu_sc as plsc

BATCH = 16384
N_COLS = 1000
K = 256
THRESHOLD = 0.5

NUM_CORES = 2
NUM_SUBCORES = 16
NUM_WORKERS = NUM_CORES * NUM_SUBCORES
ROWS_PER_WORKER = BATCH // NUM_WORKERS
CHUNK = 32
NUM_CHUNKS = ROWS_PER_WORKER // CHUNK
LANES = 16
K_VECS = K // LANES


def _sc_kernel(
    sim_hbm, idx_hbm, out_hbm,
    idx_v, rows0, rows1, out0, out1,
    sem_in0, sem_in1, sem_out0, sem_out1,
):
    wid = lax.axis_index("s") * NUM_CORES + lax.axis_index("c")
    base = wid * ROWS_PER_WORKER
    ones = jnp.full((LANES,), 1.0, jnp.float32)
    zeros = jnp.zeros((LANES,), jnp.float32)

    pltpu.sync_copy(idx_hbm, idx_v)
    idx_vecs = [idx_v[pl.ds(j * LANES, LANES)] for j in range(K_VECS)]

    rows_bufs = (rows0, rows1)
    out_bufs = (out0, out1)
    sem_in = (sem_in0, sem_in1)
    sem_out = (sem_out0, sem_out1)

    def start_in(c):
        b = c % 2
        row0 = base + c * CHUNK
        return pltpu.async_copy(
            sim_hbm.at[pl.ds(row0 * N_COLS, CHUNK * N_COLS)], rows_bufs[b], sem_in[b]
        )

    in_cp = {0: start_in(0)}
    out_cp = [None, None]
    for c in range(NUM_CHUNKS):
        b = c % 2
        if c + 1 < NUM_CHUNKS:
            in_cp[c + 1] = start_in(c + 1)
        in_cp[c].wait()
        if out_cp[b] is not None:
            out_cp[b].wait()
        rows_v = rows_bufs[b]
        out_v = out_bufs[b]

        def row_body(r, carry, rows_v=rows_v, out_v=out_v):
            rbase = jnp.full((LANES,), r * N_COLS, jnp.int32)
            for j in range(K_VECS):
                g = plsc.load_gather(rows_v, [rbase + idx_vecs[j]])
                out_v[pl.ds(r * K + j * LANES, LANES)] = jnp.where(
                    g >= THRESHOLD, ones, zeros
                )
            return carry

        lax.fori_loop(0, CHUNK, row_body, 0)
        out_cp[b] = pltpu.async_copy(
            out_v, out_hbm.at[pl.ds((base + c * CHUNK) * K, CHUNK * K)], sem_out[b]
        )
    out_cp[0].wait()
    out_cp[1].wait()


_call = functools.partial(
    pl.kernel,
    out_type=jax.ShapeDtypeStruct((BATCH * K,), jnp.float32),
    mesh=plsc.VectorSubcoreMesh(core_axis_name="c", subcore_axis_name="s"),
    compiler_params=pltpu.CompilerParams(
        use_tc_tiling_on_sc=False, needs_layout_passes=False
    ),
    scratch_types=[
        pltpu.VMEM((K,), jnp.int32),
        pltpu.VMEM((CHUNK * N_COLS,), jnp.float32),
        pltpu.VMEM((CHUNK * N_COLS,), jnp.float32),
        pltpu.VMEM((CHUNK * K,), jnp.float32),
        pltpu.VMEM((CHUNK * K,), jnp.float32),
        pltpu.SemaphoreType.DMA,
        pltpu.SemaphoreType.DMA,
        pltpu.SemaphoreType.DMA,
        pltpu.SemaphoreType.DMA,
    ],
)(_sc_kernel)


def kernel(similarities, label_indices):
    out_flat = _call(similarities.reshape(-1), label_indices)
    return out_flat.reshape(BATCH, K)


# speedup vs baseline: 1.1999x; 1.0819x over previous
import functools

import jax
import jax.numpy as jnp
from jax import lax
from jax.experimental import pallas as pl
from jax.experimental.pallas import tpu as pltpu
from jax.experimental.pallas import tpu_sc as plsc

BATCH = 16384
N_COLS = 1000
K = 256
THRESHOLD = 0.5

NUM_CORES = 2
NUM_SUBCORES = 16
NUM_WORKERS = NUM_CORES * NUM_SUBCORES
ROWS_PER_WORKER = BATCH // NUM_WORKERS
CHUNK = 32
NUM_CHUNKS = ROWS_PER_WORKER // CHUNK
LANES = 16
K_VECS = K // LANES


def _sc_kernel(
    sim_hbm, idx_hbm, out_hbm,
    idx_v, rows0, rows1, out0, out1,
    sem_in0, sem_in1, sem_out0, sem_out1,
):
    wid = lax.axis_index("s") * NUM_CORES + lax.axis_index("c")
    base = wid * ROWS_PER_WORKER
    ones = jnp.full((LANES,), 1.0, jnp.float32)
    zeros = jnp.zeros((LANES,), jnp.float32)

    pltpu.sync_copy(idx_hbm, idx_v)
    idx_vecs = [idx_v[pl.ds(j * LANES, LANES)] for j in range(K_VECS)]

    rows_bufs = (rows0, rows1)
    out_bufs = (out0, out1)
    sem_in = (sem_in0, sem_in1)
    sem_out = (sem_out0, sem_out1)

    def start_in(c):
        b = c % 2
        row0 = base + c * CHUNK
        return pltpu.async_copy(
            sim_hbm.at[pl.ds(row0 * N_COLS, CHUNK * N_COLS)], rows_bufs[b], sem_in[b]
        )

    in_cp = {0: start_in(0)}
    out_cp = [None, None]
    for c in range(NUM_CHUNKS):
        b = c % 2
        if c + 1 < NUM_CHUNKS:
            in_cp[c + 1] = start_in(c + 1)
        in_cp[c].wait()
        if out_cp[b] is not None:
            out_cp[b].wait()
        rows_v = rows_bufs[b]
        out_v = out_bufs[b]

        @plsc.parallel_loop(0, CHUNK, step=1, unroll=1)
        def row_body(r, rows_v=rows_v, out_v=out_v):
            rbase = jnp.full((LANES,), r * N_COLS, jnp.int32)
            for j in range(K_VECS):
                g = plsc.load_gather(rows_v, [rbase + idx_vecs[j]])
                out_v[pl.ds(r * K + j * LANES, LANES)] = jnp.where(
                    g >= THRESHOLD, ones, zeros
                )
        out_cp[b] = pltpu.async_copy(
            out_v, out_hbm.at[pl.ds((base + c * CHUNK) * K, CHUNK * K)], sem_out[b]
        )
    out_cp[0].wait()
    out_cp[1].wait()


_call = functools.partial(
    pl.kernel,
    out_type=jax.ShapeDtypeStruct((BATCH * K,), jnp.float32),
    mesh=plsc.VectorSubcoreMesh(core_axis_name="c", subcore_axis_name="s"),
    compiler_params=pltpu.CompilerParams(
        use_tc_tiling_on_sc=False, needs_layout_passes=False
    ),
    scratch_types=[
        pltpu.VMEM((K,), jnp.int32),
        pltpu.VMEM((CHUNK * N_COLS,), jnp.float32),
        pltpu.VMEM((CHUNK * N_COLS,), jnp.float32),
        pltpu.VMEM((CHUNK * K,), jnp.float32),
        pltpu.VMEM((CHUNK * K,), jnp.float32),
        pltpu.SemaphoreType.DMA,
        pltpu.SemaphoreType.DMA,
        pltpu.SemaphoreType.DMA,
        pltpu.SemaphoreType.DMA,
    ],
)(_sc_kernel)


def kernel(similarities, label_indices):
    out_flat = _call(similarities.reshape(-1), label_indices)
    return out_flat.reshape(BATCH, K)


# speedup vs baseline: 1.2114x; 1.0096x over previous
import functools

import jax
import jax.numpy as jnp
from jax import lax
from jax.experimental import pallas as pl
from jax.experimental.pallas import tpu as pltpu
from jax.experimental.pallas import tpu_sc as plsc

BATCH = 16384
N_COLS = 1000
K = 256
THRESHOLD = 0.5

NUM_CORES = 2
NUM_SUBCORES = 16
NUM_WORKERS = NUM_CORES * NUM_SUBCORES
ROWS_PER_WORKER = BATCH // NUM_WORKERS
CHUNK = 32
NUM_CHUNKS = ROWS_PER_WORKER // CHUNK
LANES = 16
K_VECS = K // LANES


def _sc_kernel(
    sim_hbm, idx_hbm, out_hbm,
    idx_v, rows0, rows1, out0, out1,
    sem_in0, sem_in1, sem_out0, sem_out1,
):
    wid = lax.axis_index("s") * NUM_CORES + lax.axis_index("c")
    base = wid * ROWS_PER_WORKER
    ones = jnp.full((LANES,), 1.0, jnp.float32)
    zeros = jnp.zeros((LANES,), jnp.float32)

    pltpu.sync_copy(idx_hbm, idx_v)
    idx_vecs = [idx_v[pl.ds(j * LANES, LANES)] for j in range(K_VECS)]

    rows_bufs = (rows0, rows1)
    out_bufs = (out0, out1)
    sem_in = (sem_in0, sem_in1)
    sem_out = (sem_out0, sem_out1)

    def in_slice(c):
        row0 = base + c * CHUNK
        return sim_hbm.at[pl.ds(row0 * N_COLS, CHUNK * N_COLS)]

    def out_slice(c):
        row0 = base + c * CHUNK
        return out_hbm.at[pl.ds(row0 * K, CHUNK * K)]

    pltpu.async_copy(in_slice(0), rows_bufs[0], sem_in[0])
    pltpu.async_copy(in_slice(1), rows_bufs[1], sem_in[1])

    def chunk_pair(p, carry):
        for phase in range(2):
            b = phase
            c = p * 2 + phase
            rows_v = rows_bufs[b]
            out_v = out_bufs[b]
            pltpu.make_async_copy(in_slice(0), rows_v, sem_in[b]).wait()

            @pl.when(c >= 2)
            def _():
                pltpu.make_async_copy(out_v, out_slice(0), sem_out[b]).wait()

            @plsc.parallel_loop(0, CHUNK, step=1, unroll=4)
            def row_body(r, rows_v=rows_v, out_v=out_v):
                rbase = jnp.full((LANES,), r * N_COLS, jnp.int32)
                for j in range(K_VECS):
                    g = plsc.load_gather(rows_v, [rbase + idx_vecs[j]])
                    out_v[pl.ds(r * K + j * LANES, LANES)] = jnp.where(
                        g >= THRESHOLD, ones, zeros
                    )

            pltpu.async_copy(out_v, out_slice(c), sem_out[b])

            @pl.when(c + 2 < NUM_CHUNKS)
            def _():
                pltpu.async_copy(in_slice(c + 2), rows_v, sem_in[b])

        return carry

    lax.fori_loop(0, NUM_CHUNKS // 2, chunk_pair, 0)
    pltpu.make_async_copy(out_bufs[0], out_slice(0), sem_out[0]).wait()
    pltpu.make_async_copy(out_bufs[1], out_slice(0), sem_out[1]).wait()


_call = functools.partial(
    pl.kernel,
    out_type=jax.ShapeDtypeStruct((BATCH * K,), jnp.float32),
    mesh=plsc.VectorSubcoreMesh(core_axis_name="c", subcore_axis_name="s"),
    compiler_params=pltpu.CompilerParams(
        use_tc_tiling_on_sc=False, needs_layout_passes=False
    ),
    scratch_types=[
        pltpu.VMEM((K,), jnp.int32),
        pltpu.VMEM((CHUNK * N_COLS,), jnp.float32),
        pltpu.VMEM((CHUNK * N_COLS,), jnp.float32),
        pltpu.VMEM((CHUNK * K,), jnp.float32),
        pltpu.VMEM((CHUNK * K,), jnp.float32),
        pltpu.SemaphoreType.DMA,
        pltpu.SemaphoreType.DMA,
        pltpu.SemaphoreType.DMA,
        pltpu.SemaphoreType.DMA,
    ],
)(_sc_kernel)


def kernel(similarities, label_indices):
    out_flat = _call(similarities.reshape(-1), label_indices)
    return out_flat.reshape(BATCH, K)


# speedup vs baseline: 2.2127x; 1.8266x over previous
import functools

import jax
import jax.numpy as jnp
from jax import lax
from jax.experimental import pallas as pl
from jax.experimental.pallas import tpu as pltpu
from jax.experimental.pallas import tpu_sc as plsc

BATCH = 16384
N_COLS = 1000
K = 256
THRESHOLD = 0.5

NUM_CORES = 2
NUM_SUBCORES = 16
NUM_WORKERS = NUM_CORES * NUM_SUBCORES
ROWS_PER_WORKER = BATCH // NUM_WORKERS
CHUNK = 32
NUM_CHUNKS = ROWS_PER_WORKER // CHUNK
LANES = 16
K_VECS = K // LANES


def _sc_kernel(
    sim_hbm, idx_hbm, out_hbm,
    idx_v, rows0, rows1, out0, out1,
    sem_in0, sem_in1, sem_out0, sem_out1,
):
    wid = lax.axis_index("s") * NUM_CORES + lax.axis_index("c")
    base = wid * ROWS_PER_WORKER
    ones = jnp.full((LANES,), 1.0, jnp.float32)
    zeros = jnp.zeros((LANES,), jnp.float32)

    pltpu.sync_copy(idx_hbm, idx_v)
    idx_vecs = [idx_v[pl.ds(j * LANES, LANES)] for j in range(K_VECS)]

    rows_bufs = (rows0, rows1)
    out_bufs = (out0, out1)
    sem_in = (sem_in0, sem_in1)
    sem_out = (sem_out0, sem_out1)

    def in_slice(c):
        row0 = base + c * CHUNK
        return sim_hbm.at[pl.ds(row0, CHUNK)]

    def out_slice(c):
        row0 = base + c * CHUNK
        return out_hbm.at[pl.ds(row0, CHUNK)]

    pltpu.async_copy(in_slice(0), rows_bufs[0], sem_in[0])
    pltpu.async_copy(in_slice(1), rows_bufs[1], sem_in[1])

    def chunk_pair(p, carry):
        for phase in range(2):
            b = phase
            c = p * 2 + phase
            rows_v = rows_bufs[b]
            out_v = out_bufs[b]
            pltpu.make_async_copy(in_slice(0), rows_v, sem_in[b]).wait()

            @pl.when(c >= 2)
            def _():
                pltpu.make_async_copy(out_v, out_slice(0), sem_out[b]).wait()

            @plsc.parallel_loop(0, CHUNK, step=1, unroll=4)
            def row_body(r, rows_v=rows_v, out_v=out_v):
                rvec = jnp.full((LANES,), r, jnp.int32)
                for j in range(K_VECS):
                    g = plsc.load_gather(rows_v, [rvec, idx_vecs[j]])
                    out_v[r, pl.ds(j * LANES, LANES)] = jnp.where(
                        g >= THRESHOLD, ones, zeros
                    )

            pltpu.async_copy(out_v, out_slice(c), sem_out[b])

            @pl.when(c + 2 < NUM_CHUNKS)
            def _():
                pltpu.async_copy(in_slice(c + 2), rows_v, sem_in[b])

        return carry

    lax.fori_loop(0, NUM_CHUNKS // 2, chunk_pair, 0)
    pltpu.make_async_copy(out_bufs[0], out_slice(0), sem_out[0]).wait()
    pltpu.make_async_copy(out_bufs[1], out_slice(0), sem_out[1]).wait()


_call = functools.partial(
    pl.kernel,
    out_type=jax.ShapeDtypeStruct((BATCH, K), jnp.float32),
    mesh=plsc.VectorSubcoreMesh(core_axis_name="c", subcore_axis_name="s"),
    compiler_params=pltpu.CompilerParams(
        use_tc_tiling_on_sc=True, needs_layout_passes=False
    ),
    scratch_types=[
        pltpu.VMEM((K,), jnp.int32),
        pltpu.VMEM((CHUNK, N_COLS), jnp.float32),
        pltpu.VMEM((CHUNK, N_COLS), jnp.float32),
        pltpu.VMEM((CHUNK, K), jnp.float32),
        pltpu.VMEM((CHUNK, K), jnp.float32),
        pltpu.SemaphoreType.DMA,
        pltpu.SemaphoreType.DMA,
        pltpu.SemaphoreType.DMA,
        pltpu.SemaphoreType.DMA,
    ],
)(_sc_kernel)


def kernel(similarities, label_indices):
    return _call(similarities, label_indices)


# speedup vs baseline: 4.1760x; 1.8873x over previous
import functools

import jax
import jax.numpy as jnp
from jax import lax
from jax.experimental import pallas as pl
from jax.experimental.pallas import tpu as pltpu
from jax.experimental.pallas import tpu_sc as plsc

BATCH = 16384
N_COLS = 1000
K = 256
THRESHOLD = 0.5

NUM_CORES = 2
NUM_SUBCORES = 16
NUM_WORKERS = NUM_CORES * NUM_SUBCORES
COLS_PER_WORKER = BATCH // NUM_WORKERS
W = 128
NUM_SLABS = COLS_PER_WORKER // W
LANES = 16
HALF_K = K // 2


def _sc_kernel(
    sim_hbm, idxlo_hbm, idxhi_hbm, out_hbm,
    idx_lo, idx_hi, slab0, slab1, out_v,
    sem_in0, sem_in1, sem_out,
):
    wid = lax.axis_index("s") * NUM_CORES + lax.axis_index("c")
    col_base = wid * COLS_PER_WORKER
    ones = jnp.full((LANES,), 1.0, jnp.float32)
    zeros = jnp.zeros((LANES,), jnp.float32)
    iota = lax.iota(jnp.int32, LANES)
    perms = [(iota + s) & (LANES - 1) for s in range(LANES)]

    pltpu.sync_copy(idxlo_hbm, idx_lo)
    pltpu.sync_copy(idxhi_hbm, idx_hi)

    slab_bufs = (slab0, slab1)
    sem_in = (sem_in0, sem_in1)

    def start_in(s, b):
        c0 = col_base + s * W
        lo = pltpu.async_copy(
            sim_hbm.at[idx_lo, pl.ds(c0, W)],
            slab_bufs[b].at[pl.ds(0, HALF_K)],
            sem_in[b],
        )
        hi = pltpu.async_copy(
            sim_hbm.at[idx_hi, pl.ds(c0, W)],
            slab_bufs[b].at[pl.ds(HALF_K, HALF_K)],
            sem_in[b],
        )
        return lo, hi

    in_cp = {0: start_in(0, 0), 1: start_in(1, 1)}
    out_cp = None
    for s in range(NUM_SLABS):
        b = s % 2
        lo, hi = in_cp.pop(s)
        lo.wait()
        hi.wait()
        if out_cp is not None:
            out_cp.wait()
        slab_v = slab_bufs[b]

        @plsc.parallel_loop(0, (W // LANES) * (K // LANES), step=1, unroll=2)
        def block_body(q, slab_v=slab_v):
            w0 = (q >> 4) << 4
            j0 = (q & (LANES - 1)) << 4
            jvec = jnp.full((LANES,), j0, jnp.int32) + iota
            w0vec = jnp.full((LANES,), w0, jnp.int32)
            for d in range(LANES):
                wvec = w0vec + perms[d]
                g = plsc.load_gather(slab_v, [jvec, wvec])
                v = jnp.where(g >= THRESHOLD, ones, zeros)
                plsc.store_scatter(out_v, [wvec, jvec], v)

        out_cp = pltpu.async_copy(
            out_v, out_hbm.at[pl.ds(col_base + s * W, W)], sem_out
        )
        if s + 2 < NUM_SLABS:
            in_cp[s + 2] = start_in(s + 2, b)
    out_cp.wait()


_call = functools.partial(
    pl.kernel,
    out_type=jax.ShapeDtypeStruct((BATCH, K), jnp.float32),
    mesh=plsc.VectorSubcoreMesh(core_axis_name="c", subcore_axis_name="s"),
    compiler_params=pltpu.CompilerParams(
        use_tc_tiling_on_sc=True, needs_layout_passes=False
    ),
    scratch_types=[
        pltpu.VMEM((HALF_K,), jnp.int32),
        pltpu.VMEM((HALF_K,), jnp.int32),
        pltpu.VMEM((K, W), jnp.float32),
        pltpu.VMEM((K, W), jnp.float32),
        pltpu.VMEM((W, K), jnp.float32),
        pltpu.SemaphoreType.DMA,
        pltpu.SemaphoreType.DMA,
        pltpu.SemaphoreType.DMA,
    ],
)(_sc_kernel)


def kernel(similarities, label_indices):
    return _call(
        similarities.T,
        label_indices[:HALF_K],
        label_indices[HALF_K:],
    )


# speedup vs baseline: 4.7199x; 1.1302x over previous
import functools

import jax
import jax.numpy as jnp
from jax import lax
from jax.experimental import pallas as pl
from jax.experimental.pallas import tpu as pltpu
from jax.experimental.pallas import tpu_sc as plsc

BATCH = 16384
N_COLS = 1000
K = 256
THRESHOLD = 0.5

NUM_CORES = 2
NUM_SUBCORES = 16
NUM_WORKERS = NUM_CORES * NUM_SUBCORES
COLS_PER_WORKER = BATCH // NUM_WORKERS
W = 128
NUM_SLABS = COLS_PER_WORKER // W
LANES = 16
HALF_K = K // 2


def _sc_kernel(
    sim_hbm, idxlo_hbm, idxhi_hbm, out_hbm,
    idx_lo, idx_hi, slab0, slab1, out_v,
    sem_in0, sem_in1, sem_out,
):
    wid = lax.axis_index("s") * NUM_CORES + lax.axis_index("c")
    col_base = wid * COLS_PER_WORKER
    ones = jnp.full((LANES,), 1.0, jnp.float32)
    zeros = jnp.zeros((LANES,), jnp.float32)
    iota = lax.iota(jnp.int32, LANES)
    perms = [(iota + s) & (LANES - 1) for s in range(LANES)]

    pltpu.sync_copy(idxlo_hbm, idx_lo)
    pltpu.sync_copy(idxhi_hbm, idx_hi)

    slab_bufs = (slab0, slab1)
    sem_in = (sem_in0, sem_in1)

    def start_in(s, b):
        c0 = col_base + s * W
        lo = pltpu.async_copy(
            sim_hbm.at[idx_lo, pl.ds(c0, W)],
            slab_bufs[b].at[pl.ds(0, HALF_K)],
            sem_in[b],
        )
        hi = pltpu.async_copy(
            sim_hbm.at[idx_hi, pl.ds(c0, W)],
            slab_bufs[b].at[pl.ds(HALF_K, HALF_K)],
            sem_in[b],
        )
        return lo, hi

    in_cp = {0: start_in(0, 0), 1: start_in(1, 1)}
    out_cp = None
    for s in range(NUM_SLABS):
        b = s % 2
        lo, hi = in_cp.pop(s)
        lo.wait()
        hi.wait()
        if out_cp is not None:
            out_cp.wait()
        slab_v = slab_bufs[b]

        @plsc.parallel_loop(0, (W // LANES) * (K // LANES), step=1, unroll=4)
        def block_body(q, slab_v=slab_v):
            w0 = (q >> 4) << 4
            j0 = (q & (LANES - 1)) << 4
            jvec = jnp.full((LANES,), j0, jnp.int32) + iota
            w0vec = jnp.full((LANES,), w0, jnp.int32)
            for d in range(LANES):
                wvec = w0vec + perms[d]
                g = plsc.load_gather(slab_v, [jvec, wvec])
                v = jnp.where(g >= THRESHOLD, ones, zeros)
                plsc.store_scatter(out_v, [wvec, jvec], v)

        out_cp = pltpu.async_copy(
            out_v, out_hbm.at[pl.ds(col_base + s * W, W)], sem_out
        )
        if s + 2 < NUM_SLABS:
            in_cp[s + 2] = start_in(s + 2, b)
    out_cp.wait()


_call = functools.partial(
    pl.kernel,
    out_type=jax.ShapeDtypeStruct((BATCH, K), jnp.float32),
    mesh=plsc.VectorSubcoreMesh(core_axis_name="c", subcore_axis_name="s"),
    compiler_params=pltpu.CompilerParams(
        use_tc_tiling_on_sc=True, needs_layout_passes=False
    ),
    scratch_types=[
        pltpu.VMEM((HALF_K,), jnp.int32),
        pltpu.VMEM((HALF_K,), jnp.int32),
        pltpu.VMEM((K, W), jnp.float32),
        pltpu.VMEM((K, W), jnp.float32),
        pltpu.VMEM((W, K), jnp.float32),
        pltpu.SemaphoreType.DMA,
        pltpu.SemaphoreType.DMA,
        pltpu.SemaphoreType.DMA,
    ],
)(_sc_kernel)


def kernel(similarities, label_indices):
    return _call(
        similarities.T,
        label_indices[:HALF_K],
        label_indices[HALF_K:],
    )
